# baseline (speedup 1.0000x reference)
import functools

import jax
import jax.numpy as jnp
import numpy as np
from jax import lax
from jax.experimental import pallas as pl
from jax.experimental.pallas import tpu as pltpu
from jax.experimental.pallas import tpu_sc as plsc

_INT_MIN = np.int32(-2147483648)
_INT_MAX = np.int32(2147483647)
_U_MSB = np.uint32(0x80000000)
_U_ALL = np.uint32(0xFFFFFFFF)


def _encode(x):
    i = lax.bitcast_convert_type(x, jnp.int32)
    return jnp.where(i < 0, i ^ _INT_MAX, i)


def _decode_f32(s):
    m = s ^ _INT_MIN
    f_bits = jnp.where(m < 0, m & _INT_MAX, ~m)
    return lax.bitcast_convert_type(f_bits, jnp.float32)


def _tc_mask_kernel(x_ref, pr_ref, out_ref, k_ref):
    x = x_ref[...]
    n = x.shape[1]
    k_ref[...] = _encode(x)

    pr_s = pr_ref[0, 0]
    pr_eff = jnp.where(pr_s > 10, 10, pr_s).astype(jnp.float32) * 0.1
    pr_bis = 1.0 - pr_eff
    qidx = pr_bis * jnp.float32(n - 1)
    lo_f = jnp.floor(qidx)
    frac = qidx - lo_f
    r = lo_f.astype(jnp.int32)

    def body(j, p):
        bit = 31 - j
        low_mask = (jnp.int32(1) << bit) - 1
        t_s = (p | low_mask) ^ _INT_MIN
        k = k_ref[...]
        c = jnp.sum((k <= t_s).astype(jnp.int32), axis=1, keepdims=True)
        return jnp.where(c > r, p, p | (jnp.int32(1) << bit))

    p = lax.fori_loop(0, 32, body, jnp.zeros((x.shape[0], 1), jnp.int32))

    v_lo_s = p ^ _INT_MIN
    k = k_ref[...]
    c_le = jnp.sum((k <= v_lo_s).astype(jnp.int32), axis=1, keepdims=True)
    gmin = jnp.min(jnp.where(k > v_lo_s, k, _INT_MAX), axis=1, keepdims=True)
    v_hi_s = jnp.where((c_le > r + 1) | (c_le >= n), v_lo_s, gmin)

    x_lo = _decode_f32(v_lo_s)
    x_hi = _decode_f32(v_hi_s)
    q = x_lo + (x_hi - x_lo) * frac

    res = (x >= q).astype(jnp.float32)
    out_ref[...] = jnp.where(pr_s >= 10, 1.0, jnp.where(pr_s == 0, 0.0, res))


def _run_tc(flat, pr_arr, interpret=False):
    b, n = flat.shape
    return pl.pallas_call(
        _tc_mask_kernel,
        out_shape=jax.ShapeDtypeStruct((b, n), jnp.float32),
        scratch_shapes=[pltpu.VMEM((b, n), jnp.int32)],
        interpret=interpret,
    )(flat, pr_arr)


_NB = 4
_COLS = 12288
_NCH = _COLS // 16
_NTOT = 196608


def _sc_call(x3, pi, pf, ix):
    mesh = plsc.VectorSubcoreMesh(core_axis_name="c", subcore_axis_name="s")

    @functools.partial(
        pl.kernel,
        out_type=jax.ShapeDtypeStruct((8, 16), jnp.float32),
        mesh=mesh,
        scratch_types=[
            pltpu.VMEM((_NB, _COLS), jnp.float32),
            pltpu.VMEM((_NB, _COLS), jnp.uint32),
            pltpu.VMEM((_NB, 256), jnp.int32),
            pltpu.VMEM((256,), jnp.int32),
            pltpu.VMEM((_NB, 16), jnp.int32),
            pltpu.VMEM((16, _NB, 16), jnp.int32),
            pltpu.VMEM((_NB, 16), jnp.int32),
            pltpu.VMEM((16,), jnp.int32),
            pltpu.VMEM((16,), jnp.int32),
            pltpu.VMEM((16,), jnp.float32),
            pltpu.VMEM((4,), jnp.int32),
            pltpu.SemaphoreType.DMA,
            pltpu.VMEM_SHARED((_NB, 256), jnp.int32),
            pltpu.VMEM_SHARED((_NB, 16), jnp.int32),
            pltpu.VMEM_SHARED((16, _NB, 16), jnp.int32),
        ],
        compiler_params=pltpu.CompilerParams(
            use_tc_tiling_on_sc=False, needs_layout_passes=False),
    )
    def k(x_hbm, pi_hbm, pf_hbm, ix_hbm, o_hbm, xb, ub, hist, hbuf, mbuf,
          sbuf, vbuf, wbuf, pibuf, pfbuf, idx4, ldsem, sh_hist, sh_meta,
          sh_stats):
        c = lax.axis_index("c")
        s = lax.axis_index("s")
        iota = lax.broadcasted_iota(jnp.int32, (16,), 0)
        zeros16 = jnp.zeros((16,), jnp.int32)
        ones16 = jnp.ones((16,), jnp.int32)

        lds = [pltpu.async_copy(
            x_hbm.at[pl.ds((_NB * c + i) * _NTOT + s * _COLS, _COLS)],
            xb.at[i], ldsem) for i in range(_NB)]
        pltpu.sync_copy(pi_hbm, pibuf)
        pltpu.sync_copy(pf_hbm, pfbuf)
        pltpu.sync_copy(ix_hbm, idx4)
        piv = pibuf[...]
        pfv = pfbuf[...]
        r = piv[0]
        frac = pfv[0]

        _U = 16
        _NL = _NCH // _U

        def zero_hist():
            for row in range(_NB):
                for ch in range(16):
                    hist[row, pl.ds(ch * 16, 16)] = zeros16

        def scatter_level(shift, match_shift, pfx, fuse_encode):
            for i in range(_NB):
                row16 = jnp.full((16,), i, jnp.int32)
                pfx_u = None if pfx is None else pfx[i].astype(jnp.uint32)

                def sc_body(j, carry, i=i, row16=row16, pfx_u=pfx_u):
                    base = j * (_U * 16)
                    offs = [base + kk * 16 for kk in range(_U)]
                    if fuse_encode:
                        xvs = [xb[i, pl.ds(o, 16)] for o in offs]
                        ius = [lax.bitcast_convert_type(xv, jnp.uint32)
                               for xv in xvs]
                        uvs = [jnp.where(iu >= _U_MSB, ~iu, iu | _U_MSB)
                               for iu in ius]
                        for kk in range(_U):
                            ub[i, pl.ds(offs[kk], 16)] = uvs[kk]
                    else:
                        uvs = [ub[i, pl.ds(o, 16)] for o in offs]
                    buckets = [((uv >> shift) & np.uint32(0xFF))
                               .astype(jnp.int32) for uv in uvs]
                    if pfx_u is None:
                        for kk in range(_U):
                            plsc.addupdate_scatter(
                                hist, [row16, buckets[kk]], ones16)
                    else:
                        mks = [(uv >> match_shift) == pfx_u for uv in uvs]
                        for kk in range(_U):
                            plsc.addupdate_scatter(
                                hist, [row16, buckets[kk]], ones16,
                                mask=mks[kk])
                    return carry
                lax.fori_loop(0, _NL, sc_body, 0)

        pfx = None
        cb = [jnp.int32(0)] * _NB
        nv = [jnp.int32(0)] * _NB

        for lvl in range(4):
            shift = 24 - 8 * lvl
            mshift = shift + 8 if lvl > 0 else None
            zero_hist()
            @pl.when(s == 0)
            def _():
                pltpu.sync_copy(hist, sh_hist)
            plsc.subcore_barrier()

            if lvl == 0:
                for d in lds:
                    d.wait()
            scatter_level(shift, mshift, pfx, fuse_encode=(lvl == 0))
            pltpu.sync_copy(hist, sh_hist.at[idx4], add=True)
            plsc.subcore_barrier()

            b = s & 3
            pltpu.sync_copy(sh_hist.at[b], hbuf)
            cb_b = jnp.int32(0)
            for i in range(_NB):
                cb_b = jnp.where(b == i, cb[i], cb_b)
            want = r - cb_b

            def search_body(ch, carry):
                run, found, bucket, cbl = carry
                v = hbuf[pl.ds(ch * 16, 16)]
                tot = jnp.sum(v)
                cond = (found == 0) & (run + tot > want)
                bucket = jnp.where(cond, ch, bucket)
                cbl = jnp.where(cond, run, cbl)
                found = found | cond.astype(jnp.int32)
                return run + tot, found, bucket, cbl

            init = (jnp.int32(0), jnp.int32(0), jnp.int32(0), jnp.int32(0))
            _, _, bucket, cbl = lax.fori_loop(0, 16, search_body, init)
            vch = hbuf[pl.ds(bucket * 16, 16)]
            csum = jnp.cumsum(vch)
            below = cbl + csum - vch
            lane_hit = (below <= want) & (below + vch > want)
            lane = jnp.sum(jnp.where(lane_hit, iota, 0))
            cbl2 = jnp.sum(jnp.where(lane_hit, below, 0))
            nvl2 = jnp.sum(jnp.where(lane_hit, vch, 0))
            bfull = bucket * 16 + lane

            if pfx is not None:
                pfx_b = jnp.int32(0)
                for i in range(_NB):
                    pfx_b = jnp.where(b == i, pfx[i], pfx_b)
                bfull = (pfx_b << 8) | bfull

            mv = ((iota == 0).astype(jnp.int32) * bfull
                  + (iota == 1).astype(jnp.int32) * cbl2
                  + (iota == 2).astype(jnp.int32) * nvl2)
            wbuf[...] = mv

            @pl.when(s < 4)
            def _():
                pltpu.sync_copy(wbuf, sh_meta.at[s])
            plsc.subcore_barrier()
            pltpu.sync_copy(sh_meta, mbuf)
            mrows = [mbuf[i, :] for i in range(_NB)]
            pfx = [mrows[i][0] for i in range(_NB)]
            cb = [cb[i] + mrows[i][1] for i in range(_NB)]
            nv = [mrows[i][2] for i in range(_NB)]


        for i in range(_NB):
            v_u = lax.bitcast_convert_type(
                jnp.full((16,), pfx[i], jnp.int32), jnp.uint32)

            def st_body(j, accs, i=i, v_u=v_u):
                base = j * (_U * 16)
                uvs = [ub[i, pl.ds(base + kk * 16, 16)] for kk in range(_U)]
                cands = [jnp.where(uv > v_u, uv, _U_ALL) for uv in uvs]
                return tuple(jnp.minimum(accs[kk], cands[kk])
                             for kk in range(_U))

            acc0 = tuple(jnp.full((16,), _U_ALL, jnp.uint32)
                         for _ in range(_U))
            accs = lax.fori_loop(0, _NL, st_body, acc0)
            accf = accs[0]
            for kk in range(1, _U):
                accf = jnp.minimum(accf, accs[kk])
            mn_s_vec = lax.bitcast_convert_type(accf ^ _U_MSB, jnp.int32)
            mn_s = jnp.min(mn_s_vec)
            vbuf[i, :] = (iota == 0).astype(jnp.int32) * mn_s \
                + (iota != 0).astype(jnp.int32) * _INT_MAX

        pltpu.sync_copy(vbuf, sh_stats.at[s])
        plsc.subcore_barrier()
        pltpu.sync_copy(sh_stats, sbuf)

        for i in range(_NB):
            mn_s = jnp.int32(_INT_MAX)
            for w in range(16):
                srow = sbuf[w, i, :]
                mn_s = jnp.minimum(mn_s, srow[0])
            v_lo_s = pfx[i] ^ _INT_MIN
            c_le = cb[i] + nv[i]
            v_hi_s = jnp.where((c_le > r + 1) | (c_le >= _NTOT),
                               v_lo_s, mn_s)
            lo_vec = jnp.full((16,), v_lo_s, jnp.int32)
            hi_vec = jnp.full((16,), v_hi_s, jnp.int32)
            x_lo = _decode_f32(lo_vec)
            x_hi = _decode_f32(hi_vec)
            qv = x_lo + (x_hi - x_lo) * frac

            @pl.when(s == i)
            def _(i=i, qv=qv):
                pfbuf[...] = qv
                pltpu.sync_copy(pfbuf, o_hbm.at[_NB * c + i])

    return k(x3, pi, pf, ix)


def _tc_mask(y4, q_rows, pr_arr):
    def mask_kernel(x_ref, q_ref, pr_ref, out_ref):
        pr_s = pr_ref[0, 0]
        q = q_ref[pl.program_id(0), 0]
        res = (x_ref[...] >= q).astype(jnp.float32)
        out_ref[...] = jnp.where(pr_s >= 10, 1.0,
                                 jnp.where(pr_s == 0, 0.0, res))

    b, w, h, ch = y4.shape
    blk = (1, w, h, ch)
    return pl.pallas_call(
        mask_kernel,
        grid=(b,),
        in_specs=[
            pl.BlockSpec(blk, lambda i: (i, 0, 0, 0)),
            pl.BlockSpec((8, 16), lambda i: (0, 0)),
            pl.BlockSpec((1, 1), lambda i: (0, 0)),
        ],
        out_specs=pl.BlockSpec(blk, lambda i: (i, 0, 0, 0)),
        out_shape=jax.ShapeDtypeStruct(y4.shape, jnp.float32),
    )(y4, q_rows, pr_arr)


def kernel(scale, pr):
    bs, ch, w, h = scale.shape
    n = ch * w * h
    y4 = jnp.transpose(scale, (0, 2, 3, 1))
    flat1 = y4.reshape(bs * n)
    pr_i = jnp.asarray(pr, jnp.int32)
    pr_eff = jnp.where(pr_i > 10, 10, pr_i).astype(jnp.float32) * 0.1
    pr_bis = 1.0 - pr_eff
    qidx = pr_bis * jnp.float32(n - 1)
    lo_f = jnp.floor(qidx)
    frac = qidx - lo_f
    r = lo_f.astype(jnp.int32)
    pi = (jnp.zeros((16,), jnp.int32).at[0].set(r)
          .at[8:12].set(jnp.arange(4, dtype=jnp.int32)))
    pf = jnp.zeros((16,), jnp.float32).at[0].set(frac)
    ix = jnp.arange(4, dtype=jnp.int32)
    q_rows = _sc_call(flat1, pi, pf, ix)
    pr_arr = pr_i.reshape(1, 1)
    out_t = _tc_mask(y4, q_rows, pr_arr)
    return jnp.transpose(out_t, (0, 3, 1, 2))


# speedup vs baseline: 1.0185x; 1.0185x over previous
import functools

import jax
import jax.numpy as jnp
import numpy as np
from jax import lax
from jax.experimental import pallas as pl
from jax.experimental.pallas import tpu as pltpu
from jax.experimental.pallas import tpu_sc as plsc

_INT_MIN = np.int32(-2147483648)
_INT_MAX = np.int32(2147483647)
_U_MSB = np.uint32(0x80000000)
_U_ALL = np.uint32(0xFFFFFFFF)


def _decode_f32(s):
    m = s ^ _INT_MIN
    f_bits = jnp.where(m < 0, m & _INT_MAX, ~m)
    return lax.bitcast_convert_type(f_bits, jnp.float32)


_NB = 4
_COLS = 12288
_NCH = _COLS // 16
_NTOT = 196608


def _sc_call(x3, pi, pf, ix):
    mesh = plsc.VectorSubcoreMesh(core_axis_name="c", subcore_axis_name="s")

    @functools.partial(
        pl.kernel,
        out_type=jax.ShapeDtypeStruct((8, 16), jnp.float32),
        mesh=mesh,
        scratch_types=[
            pltpu.VMEM((_NB, _COLS), jnp.float32),
            pltpu.VMEM((_NB, _COLS), jnp.uint32),
            pltpu.VMEM((_NB, 256), jnp.int32),
            pltpu.VMEM((256,), jnp.int32),
            pltpu.VMEM((_NB, 16), jnp.int32),
            pltpu.VMEM((16, _NB, 16), jnp.int32),
            pltpu.VMEM((_NB, 16), jnp.int32),
            pltpu.VMEM((16,), jnp.int32),
            pltpu.VMEM((16,), jnp.int32),
            pltpu.VMEM((16,), jnp.float32),
            pltpu.VMEM((4,), jnp.int32),
            pltpu.SemaphoreType.DMA,
            pltpu.VMEM_SHARED((_NB, 256), jnp.int32),
            pltpu.VMEM_SHARED((_NB, 16), jnp.int32),
            pltpu.VMEM_SHARED((16, _NB, 16), jnp.int32),
        ],
        compiler_params=pltpu.CompilerParams(
            use_tc_tiling_on_sc=False, needs_layout_passes=False),
    )
    def k(x_hbm, pi_hbm, pf_hbm, ix_hbm, o_hbm, xb, ub, hist, hbuf, mbuf,
          sbuf, vbuf, wbuf, pibuf, pfbuf, idx4, ldsem, sh_hist, sh_meta,
          sh_stats):
        c = lax.axis_index("c")
        s = lax.axis_index("s")
        iota = lax.broadcasted_iota(jnp.int32, (16,), 0)
        zeros16 = jnp.zeros((16,), jnp.int32)
        ones16 = jnp.ones((16,), jnp.int32)

        lds = [pltpu.async_copy(
            x_hbm.at[pl.ds((_NB * c + i) * _NTOT + s * _COLS, _COLS)],
            xb.at[i], ldsem) for i in range(_NB)]
        pltpu.sync_copy(pi_hbm, pibuf)
        pltpu.sync_copy(pf_hbm, pfbuf)
        pltpu.sync_copy(ix_hbm, idx4)
        piv = pibuf[...]
        pfv = pfbuf[...]
        r = piv[0]
        frac = pfv[0]

        _U = 16
        _NL = _NCH // _U

        def zero_hist():
            for row in range(_NB):
                for ch in range(16):
                    hist[row, pl.ds(ch * 16, 16)] = zeros16

        def scatter_level(shift, match_shift, pfx, fuse_encode):
            for i in range(_NB):
                row16 = jnp.full((16,), i, jnp.int32)
                pfx_u = None if pfx is None else pfx[i].astype(jnp.uint32)

                def sc_body(j, carry, i=i, row16=row16, pfx_u=pfx_u):
                    base = j * (_U * 16)
                    offs = [base + kk * 16 for kk in range(_U)]
                    if fuse_encode:
                        xvs = [xb[i, pl.ds(o, 16)] for o in offs]
                        ius = [lax.bitcast_convert_type(xv, jnp.uint32)
                               for xv in xvs]
                        uvs = [jnp.where(iu >= _U_MSB, ~iu, iu | _U_MSB)
                               for iu in ius]
                        for kk in range(_U):
                            ub[i, pl.ds(offs[kk], 16)] = uvs[kk]
                    else:
                        uvs = [ub[i, pl.ds(o, 16)] for o in offs]
                    buckets = [((uv >> shift) & np.uint32(0xFF))
                               .astype(jnp.int32) for uv in uvs]
                    if pfx_u is None:
                        for kk in range(_U):
                            plsc.addupdate_scatter(
                                hist, [row16, buckets[kk]], ones16)
                    else:
                        mks = [(uv >> match_shift) == pfx_u for uv in uvs]
                        for kk in range(_U):
                            plsc.addupdate_scatter(
                                hist, [row16, buckets[kk]], ones16,
                                mask=mks[kk])
                    return carry
                lax.fori_loop(0, _NL, sc_body, 0)

        pfx = None
        cb = [jnp.int32(0)] * _NB
        nv = [jnp.int32(0)] * _NB

        for lvl in range(4):
            shift = 24 - 8 * lvl
            mshift = shift + 8 if lvl > 0 else None
            zero_hist()
            @pl.when(s == 0)
            def _():
                pltpu.sync_copy(hist, sh_hist)
            plsc.subcore_barrier()

            if lvl == 0:
                for d in lds:
                    d.wait()
            scatter_level(shift, mshift, pfx, fuse_encode=(lvl == 0))
            pltpu.sync_copy(hist, sh_hist.at[idx4], add=True)
            plsc.subcore_barrier()

            b = s & 3
            pltpu.sync_copy(sh_hist.at[b], hbuf)
            cb_b = jnp.int32(0)
            for i in range(_NB):
                cb_b = jnp.where(b == i, cb[i], cb_b)
            want = r - cb_b

            def search_body(ch, carry):
                run, found, bucket, cbl = carry
                v = hbuf[pl.ds(ch * 16, 16)]
                tot = jnp.sum(v)
                cond = (found == 0) & (run + tot > want)
                bucket = jnp.where(cond, ch, bucket)
                cbl = jnp.where(cond, run, cbl)
                found = found | cond.astype(jnp.int32)
                return run + tot, found, bucket, cbl

            init = (jnp.int32(0), jnp.int32(0), jnp.int32(0), jnp.int32(0))
            _, _, bucket, cbl = lax.fori_loop(0, 16, search_body, init)
            vch = hbuf[pl.ds(bucket * 16, 16)]
            csum = jnp.cumsum(vch)
            below = cbl + csum - vch
            lane_hit = (below <= want) & (below + vch > want)
            lane = jnp.sum(jnp.where(lane_hit, iota, 0))
            cbl2 = jnp.sum(jnp.where(lane_hit, below, 0))
            nvl2 = jnp.sum(jnp.where(lane_hit, vch, 0))
            bfull = bucket * 16 + lane

            if pfx is not None:
                pfx_b = jnp.int32(0)
                for i in range(_NB):
                    pfx_b = jnp.where(b == i, pfx[i], pfx_b)
                bfull = (pfx_b << 8) | bfull

            mv = ((iota == 0).astype(jnp.int32) * bfull
                  + (iota == 1).astype(jnp.int32) * cbl2
                  + (iota == 2).astype(jnp.int32) * nvl2)
            wbuf[...] = mv

            @pl.when(s < 4)
            def _():
                pltpu.sync_copy(wbuf, sh_meta.at[s])
            plsc.subcore_barrier()
            pltpu.sync_copy(sh_meta, mbuf)
            mrows = [mbuf[i, :] for i in range(_NB)]
            pfx = [mrows[i][0] for i in range(_NB)]
            cb = [cb[i] + mrows[i][1] for i in range(_NB)]
            nv = [mrows[i][2] for i in range(_NB)]


        for i in range(_NB):
            v_u = lax.bitcast_convert_type(
                jnp.full((16,), pfx[i], jnp.int32), jnp.uint32)

            def st_body(j, accs, i=i, v_u=v_u):
                base = j * (_U * 16)
                uvs = [ub[i, pl.ds(base + kk * 16, 16)] for kk in range(_U)]
                cands = [jnp.where(uv > v_u, uv, _U_ALL) for uv in uvs]
                return tuple(jnp.minimum(accs[kk], cands[kk])
                             for kk in range(_U))

            acc0 = tuple(jnp.full((16,), _U_ALL, jnp.uint32)
                         for _ in range(_U))
            accs = lax.fori_loop(0, _NL, st_body, acc0)
            accf = accs[0]
            for kk in range(1, _U):
                accf = jnp.minimum(accf, accs[kk])
            mn_s_vec = lax.bitcast_convert_type(accf ^ _U_MSB, jnp.int32)
            mn_s = jnp.min(mn_s_vec)
            vbuf[i, :] = (iota == 0).astype(jnp.int32) * mn_s \
                + (iota != 0).astype(jnp.int32) * _INT_MAX

        pltpu.sync_copy(vbuf, sh_stats.at[s])
        plsc.subcore_barrier()
        pltpu.sync_copy(sh_stats, sbuf)

        for i in range(_NB):
            mn_s = jnp.int32(_INT_MAX)
            for w in range(16):
                srow = sbuf[w, i, :]
                mn_s = jnp.minimum(mn_s, srow[0])
            v_lo_s = pfx[i] ^ _INT_MIN
            c_le = cb[i] + nv[i]
            v_hi_s = jnp.where((c_le > r + 1) | (c_le >= _NTOT),
                               v_lo_s, mn_s)
            lo_vec = jnp.full((16,), v_lo_s, jnp.int32)
            hi_vec = jnp.full((16,), v_hi_s, jnp.int32)
            x_lo = _decode_f32(lo_vec)
            x_hi = _decode_f32(hi_vec)
            qv = x_lo + (x_hi - x_lo) * frac

            @pl.when(s == i)
            def _(i=i, qv=qv):
                pfbuf[...] = qv
                pltpu.sync_copy(pfbuf, o_hbm.at[_NB * c + i])

    return k(x3, pi, pf, ix)


def _tc_mask(y4, q_rows, pr_arr):
    def mask_kernel(x_ref, q_ref, pr_ref, out_ref):
        pr_s = pr_ref[0, 0]
        q = q_ref[:, 0:1].reshape(x_ref.shape[0], 1, 1, 1)
        res = (x_ref[...] >= q).astype(jnp.float32)
        out_ref[...] = jnp.where(pr_s >= 10, 1.0,
                                 jnp.where(pr_s == 0, 0.0, res))

    return pl.pallas_call(
        mask_kernel,
        out_shape=jax.ShapeDtypeStruct(y4.shape, jnp.float32),
    )(y4, q_rows, pr_arr)


def kernel(scale, pr):
    bs, ch, w, h = scale.shape
    n = ch * w * h
    y4 = jnp.transpose(scale, (0, 2, 3, 1))
    flat1 = y4.reshape(bs * n)
    pr_i = jnp.asarray(pr, jnp.int32)
    pr_eff = jnp.where(pr_i > 10, 10, pr_i).astype(jnp.float32) * 0.1
    pr_bis = 1.0 - pr_eff
    qidx = pr_bis * jnp.float32(n - 1)
    lo_f = jnp.floor(qidx)
    frac = qidx - lo_f
    r = lo_f.astype(jnp.int32)
    pi = (jnp.zeros((16,), jnp.int32).at[0].set(r)
          .at[8:12].set(jnp.arange(4, dtype=jnp.int32)))
    pf = jnp.zeros((16,), jnp.float32).at[0].set(frac)
    ix = jnp.arange(4, dtype=jnp.int32)
    q_rows = _sc_call(flat1, pi, pf, ix)
    pr_arr = pr_i.reshape(1, 1)
    out_t = _tc_mask(y4, q_rows, pr_arr)
    return jnp.transpose(out_t, (0, 3, 1, 2))
